# baseline (speedup 1.0000x reference)
import functools

import jax
import jax.numpy as jnp
from jax import lax
from jax.experimental import pallas as pl
from jax.experimental.pallas import tpu as pltpu
from jax.experimental.pallas import tpu_sc as plsc

B = 1024
N = 100000
Q = 0.7
K = 0.5

NC, NS, L = 2, 16, 16
NW = NC * NS

B_TC = 256
B_SC = B - B_TC
SCHUNK = 256
RPW = SCHUNK // NW

RB = 16
NRB = B_TC // RB
NBUF = 4
NGRP = NRB // NBUF

GR = 8
NG = RPW // GR
CW = 1280
NCH = 78
NTAIL = N - NCH * CW
TCOL = NCH * CW
CSTEP = CW // L

BPW = B // NW


@functools.lru_cache(maxsize=None)
def _sc_gather_build():
    mesh = plsc.VectorSubcoreMesh(core_axis_name="c", subcore_axis_name="s")

    @functools.partial(
        pl.kernel,
        mesh=mesh,
        out_type=[
            jax.ShapeDtypeStruct((B,), jnp.int32),
            jax.ShapeDtypeStruct((B,), jnp.float32),
        ],
        scratch_types=[
            pltpu.VMEM((BPW,), jnp.int32),
            pltpu.VMEM((BPW,), jnp.int32),
            pltpu.VMEM((BPW,), jnp.float32),
            pltpu.SemaphoreType.DMA,
        ],
    )
    def sc_gather(idx_hbm, lab_hbm, w_hbm, y_out, wb_out,
                  idx_v, y_v, w_v, sem):
        wid = lax.axis_index("s") * NC + lax.axis_index("c")
        base = wid * BPW
        pltpu.sync_copy(idx_hbm.at[pl.ds(base, BPW)], idx_v)
        pltpu.async_copy(lab_hbm.at[idx_v], y_v, sem).wait()
        pltpu.async_copy(w_hbm.at[idx_v], w_v, sem).wait()
        pltpu.sync_copy(y_v, y_out.at[pl.ds(base, BPW)])
        pltpu.sync_copy(w_v, wb_out.at[pl.ds(base, BPW)])

    return sc_gather


@functools.lru_cache(maxsize=None)
def _sc_stream_build(row0):
    mesh = plsc.VectorSubcoreMesh(core_axis_name="c", subcore_axis_name="s")

    @functools.partial(
        pl.kernel,
        mesh=mesh,
        out_type=[
            jax.ShapeDtypeStruct((SCHUNK, L), jnp.float32),
            jax.ShapeDtypeStruct((SCHUNK, L), jnp.float32),
        ],
        scratch_types=[
            pltpu.VMEM((N,), jnp.int32),
            pltpu.VMEM((GR, CW), jnp.float32),
            pltpu.VMEM((GR, CW), jnp.float32),
            pltpu.VMEM((RPW, L), jnp.int32),
            pltpu.VMEM((RPW, L), jnp.int32),
            pltpu.VMEM((RPW, L), jnp.float32),
            pltpu.VMEM((RPW, L), jnp.float32),
            pltpu.SemaphoreType.DMA,
            pltpu.SemaphoreType.DMA,
        ],
    )
    def sc_stream(idxr_hbm, yr_hbm, lab_hbm, x_hbm, p_out, z_out,
                  labv, xb0, xb1, idxb_v, yb_v, p_v, z_v, sem0, sem1):
        wid = lax.axis_index("s") * NC + lax.axis_index("c")
        rloc = wid * RPW
        pltpu.sync_copy(lab_hbm, labv)
        pltpu.sync_copy(idxr_hbm.at[pl.ds(row0 + rloc, RPW), :], idxb_v)
        pltpu.sync_copy(yr_hbm.at[pl.ds(row0 + rloc, RPW), :], yb_v)
        lane = lax.iota(jnp.int32, L)

        bufs = (xb0, xb1)
        sems = (sem0, sem1)

        def _copy(g, c, s):
            rg = pl.multiple_of(rloc + g * GR, GR)
            cc = pl.multiple_of(c * CW, CW)
            return pltpu.make_async_copy(
                x_hbm.at[pl.ds(rg, GR), pl.ds(cc, CW)], bufs[s], sems[s])

        for g in range(NG):
            idx_r = [idxb_v[g * GR + j, :] for j in range(GR)]
            y_r = [yb_v[g * GR + j, :] for j in range(GR)]
            _copy(g, 0, 0).start()
            _copy(g, 1, 1).start()

            def chunk(c, s, acc):
                za, pa = acc
                _copy(g, c, s).wait()
                buf = bufs[s]
                cbase = c * CW

                def step(it, acc2):
                    za2, pa2 = acc2
                    o = it * L
                    lv = labv[pl.ds(cbase + o, L)]
                    colv = lane + (cbase + o)
                    za3, pa3 = [], []
                    for j in range(GR):
                        e = jnp.exp(buf[j, pl.ds(o, L)])
                        e = jnp.where(colv == idx_r[j], 0.0, e)
                        m = lv == y_r[j]
                        za3.append(za2[j] + e)
                        pa3.append(pa2[j] + jnp.where(m, e, 0.0))
                    return tuple(za3), tuple(pa3)

                za, pa = lax.fori_loop(0, CSTEP, step, (za, pa))

                @pl.when(c + 2 < NCH)
                def _():
                    _copy(g, c + 2, s).start()

                return za, pa

            def pair(p_i, acc):
                return chunk(2 * p_i + 1, 1, chunk(2 * p_i, 0, acc))

            zero = jnp.zeros((L,), jnp.float32)
            za, pa = lax.fori_loop(0, NCH // 2, pair,
                                   ((zero,) * GR, (zero,) * GR))
            for j in range(GR):
                p_v[g * GR + j, :] = pa[j]
                z_v[g * GR + j, :] = za[j]

        pltpu.sync_copy(p_v, p_out.at[pl.ds(rloc, RPW), :])
        pltpu.sync_copy(z_v, z_out.at[pl.ds(rloc, RPW), :])

    return sc_stream


def _tc_body(x_hbm, labb, y, idxb, out_a, xbufs, sems):
    col = lax.broadcasted_iota(jnp.int32, (1, N), 1)
    lab = labb[...]

    def _copy(g, b):
        return pltpu.make_async_copy(
            x_hbm.at[pl.ds(pl.multiple_of(g * RB, RB), RB), :],
            xbufs.at[b], sems.at[b])

    for b in range(NBUF):
        _copy(b, b).start()

    def _block(g, b, a_s):
        _copy(g, b).wait()
        e = jnp.exp(xbufs[b])
        rows = pl.ds(pl.multiple_of(g * RB, RB), RB)
        e = jnp.where(col == idxb[rows, :], 0.0, e)
        m = lab == y[rows, :]
        z = jnp.sum(e, axis=1, keepdims=True)
        p = jnp.sum(jnp.where(m, e, 0.0), axis=1, keepdims=True)

        @pl.when(g + NBUF < NRB)
        def _():
            _copy(g + NBUF, b).start()

        prob = p / z
        a = (1.0 - prob ** Q) / Q
        return a_s + jnp.sum(a)

    def _group(grp, a_s):
        for b in range(NBUF):
            a_s = _block(grp * NBUF + b, b, a_s)
        return a_s

    out_a[0, 0] = lax.fori_loop(0, NGRP, _group, 0.0)


_tc_call = pl.pallas_call(
    _tc_body,
    in_specs=[
        pl.BlockSpec(memory_space=pltpu.MemorySpace.HBM),
        pl.BlockSpec((1, N), lambda: (0, 0)),
        pl.BlockSpec((B, 1), lambda: (0, 0)),
        pl.BlockSpec((B, 1), lambda: (0, 0)),
    ],
    out_specs=pl.BlockSpec(memory_space=pltpu.SMEM),
    out_shape=jax.ShapeDtypeStruct((1, 1), jnp.float32),
    scratch_shapes=[
        pltpu.VMEM((NBUF, RB, N), jnp.float32),
        pltpu.SemaphoreType.DMA((NBUF,)),
    ],
)


def _fin_body(a_tc, psc, zsc, wb, xt, labt, yt, idxt, out):
    e = jnp.exp(xt[...])
    colt = TCOL + lax.broadcasted_iota(jnp.int32, (1, NTAIL), 1)
    e = jnp.where(colt == idxt[...], 0.0, e)
    m = labt[...] == yt[...]
    z0 = jnp.sum(zsc[...], axis=1, keepdims=True)
    p0 = jnp.sum(psc[...], axis=1, keepdims=True)
    z = z0 + jnp.sum(e, axis=1, keepdims=True)
    p = p0 + jnp.sum(jnp.where(m, e, 0.0), axis=1, keepdims=True)
    prob = p / z
    a = (1.0 - prob ** Q) / Q
    a_total = a_tc[0, 0] + jnp.sum(a)
    mean_w = jnp.sum(wb[...]) * (1.0 / B)
    out[0, 0] = (a_total * (1.0 / B)) * mean_w - ((1.0 - K ** Q) / Q) * mean_w


_fin_call = pl.pallas_call(
    _fin_body,
    in_specs=[
        pl.BlockSpec(memory_space=pltpu.SMEM),
        pl.BlockSpec((B_SC, L), lambda: (0, 0)),
        pl.BlockSpec((B_SC, L), lambda: (0, 0)),
        pl.BlockSpec((B, 1), lambda: (0, 0)),
        pl.BlockSpec((B_SC, NTAIL), lambda: (0, 0)),
        pl.BlockSpec((1, NTAIL), lambda: (0, 0)),
        pl.BlockSpec((B_SC, 1), lambda: (0, 0)),
        pl.BlockSpec((B_SC, 1), lambda: (0, 0)),
    ],
    out_specs=pl.BlockSpec(memory_space=pltpu.SMEM),
    out_shape=jax.ShapeDtypeStruct((1, 1), jnp.float32),
)


def kernel(x, indexes, labels, weights):
    idx = indexes.astype(jnp.int32)
    lab = labels.astype(jnp.int32)
    y, wb = _sc_gather_build()(idx, lab, weights.reshape(-1))
    idx_rep = jnp.broadcast_to(idx[:, None], (B, L))
    y_rep = jnp.broadcast_to(y[:, None], (B, L))
    ps, zs = [], []
    for row0 in range(B_TC, B, SCHUNK):
        xs = lax.slice(x, (row0, 0), (row0 + SCHUNK, N))
        p_i, z_i = _sc_stream_build(row0)(idx_rep, y_rep, lab, xs)
        ps.append(p_i)
        zs.append(z_i)
    x_tc = lax.slice(x, (0, 0), (B_TC, N))
    a_tc = _tc_call(x_tc, lab.reshape(1, N), y.reshape(B, 1),
                    idx.reshape(B, 1))
    psc = jnp.concatenate(ps, axis=0)
    zsc = jnp.concatenate(zs, axis=0)
    xt = lax.slice(x, (B_TC, TCOL), (B, N))
    labt = lax.slice(lab, (TCOL,), (N,)).reshape(1, NTAIL)
    yt = lax.slice(y, (B_TC,), (B,)).reshape(B_SC, 1)
    idxt = lax.slice(idx, (B_TC,), (B,)).reshape(B_SC, 1)
    loss = _fin_call(a_tc, psc, zsc,
                     wb.reshape(B, 1), xt, labt, yt, idxt)
    return loss[0, 0]


# speedup vs baseline: 1.4527x; 1.4527x over previous
import functools

import jax
import jax.numpy as jnp
from jax import lax
from jax.experimental import pallas as pl
from jax.experimental.pallas import tpu as pltpu
from jax.experimental.pallas import tpu_sc as plsc

B = 1024
N = 100000
Q = 0.7
K = 0.5

RB = 16
NRB = B // RB
NBUF = 4
NGRP = NRB // NBUF

NC, NS, L = 2, 16, 16
NW = NC * NS
BPW = B // NW


@functools.lru_cache(maxsize=None)
def _sc_gather_build():
    mesh = plsc.VectorSubcoreMesh(core_axis_name="c", subcore_axis_name="s")

    @functools.partial(
        pl.kernel,
        mesh=mesh,
        out_type=[
            jax.ShapeDtypeStruct((B,), jnp.int32),
            jax.ShapeDtypeStruct((B,), jnp.float32),
        ],
        scratch_types=[
            pltpu.VMEM((BPW,), jnp.int32),
            pltpu.VMEM((BPW,), jnp.int32),
            pltpu.VMEM((BPW,), jnp.float32),
            pltpu.SemaphoreType.DMA,
        ],
    )
    def sc_gather(idx_hbm, lab_hbm, w_hbm, y_out, wb_out,
                  idx_v, y_v, w_v, sem):
        wid = lax.axis_index("s") * NC + lax.axis_index("c")
        base = wid * BPW
        pltpu.sync_copy(idx_hbm.at[pl.ds(base, BPW)], idx_v)
        pltpu.async_copy(lab_hbm.at[idx_v], y_v, sem).wait()
        pltpu.async_copy(w_hbm.at[idx_v], w_v, sem).wait()
        pltpu.sync_copy(y_v, y_out.at[pl.ds(base, BPW)])
        pltpu.sync_copy(w_v, wb_out.at[pl.ds(base, BPW)])

    return sc_gather


def _tc_body(x_hbm, labb, y, idxb, wb, out, xbufs, sems):
    col = lax.broadcasted_iota(jnp.int32, (1, N), 1)
    lab = labb[...]

    def _copy(g, b):
        return pltpu.make_async_copy(
            x_hbm.at[pl.ds(g * RB, RB), :], xbufs.at[b], sems.at[b])

    for b in range(NBUF):
        _copy(b, b).start()

    def _block(g, b, a_s, w_s):
        _copy(g, b).wait()
        e = jnp.exp(xbufs[b])
        rows = pl.ds(g * RB, RB)
        e = jnp.where(col == idxb[rows, :], 0.0, e)
        m = lab == y[rows, :]
        z = jnp.sum(e, axis=1, keepdims=True)
        p = jnp.sum(jnp.where(m, e, 0.0), axis=1, keepdims=True)

        @pl.when(g + NBUF < NRB)
        def _():
            _copy(g + NBUF, b).start()

        prob = p / z
        a = (1.0 - prob ** Q) / Q
        return a_s + jnp.sum(a), w_s + jnp.sum(wb[rows, :])

    def _group(grp, carry):
        a_s, w_s = carry
        for b in range(NBUF):
            a_s, w_s = _block(grp * NBUF + b, b, a_s, w_s)
        return a_s, w_s

    a_s, w_s = lax.fori_loop(0, NGRP, _group, (0.0, 0.0))
    mean_w = w_s * (1.0 / B)
    out[0, 0] = (a_s * (1.0 / B)) * mean_w - ((1.0 - K ** Q) / Q) * mean_w


_tc_call = pl.pallas_call(
    _tc_body,
    in_specs=[
        pl.BlockSpec(memory_space=pltpu.MemorySpace.HBM),
        pl.BlockSpec((1, N), lambda: (0, 0)),
        pl.BlockSpec((B, 1), lambda: (0, 0)),
        pl.BlockSpec((B, 1), lambda: (0, 0)),
        pl.BlockSpec((B, 1), lambda: (0, 0)),
    ],
    out_specs=pl.BlockSpec(memory_space=pltpu.SMEM),
    out_shape=jax.ShapeDtypeStruct((1, 1), jnp.float32),
    scratch_shapes=[
        pltpu.VMEM((NBUF, RB, N), jnp.float32),
        pltpu.SemaphoreType.DMA((NBUF,)),
    ],
)


def kernel(x, indexes, labels, weights):
    idx = indexes.astype(jnp.int32)
    lab = labels.astype(jnp.int32)
    y, wb = _sc_gather_build()(idx, lab, weights.reshape(-1))
    loss = _tc_call(x, lab.reshape(1, N), y.reshape(B, 1),
                    idx.reshape(B, 1), wb.reshape(B, 1))
    return loss[0, 0]
